# baseline (speedup 1.0000x reference)
import functools

import jax
import jax.numpy as jnp
from jax import lax
from jax.experimental import pallas as pl
from jax.experimental.pallas import tpu as pltpu
from jax.experimental.pallas import tpu_sc as plsc

B = 4
N = 8192
F = 1024
NC = 2
NS = 16
NW = NC * NS
ROWS_PER_W = N // NW
C = 32
NK = ROWS_PER_W // C

_MESH = plsc.VectorSubcoreMesh(core_axis_name="c", subcore_axis_name="s")


@functools.partial(
    pl.kernel,
    mesh=_MESH,
    out_type=jax.ShapeDtypeStruct((B, N, F), jnp.float32),
    scratch_types=[
        pltpu.VMEM((2, C, F), jnp.float32),
        pltpu.SemaphoreType.DMA,
        pltpu.SemaphoreType.DMA,
    ],
)
def _sc_broadcast(table_hbm, out_hbm, buf, sem_in, sem_out):
    wid = lax.axis_index("s") * NC + lax.axis_index("c")
    base = wid * ROWS_PER_W

    pltpu.async_copy(table_hbm.at[pl.ds(base, C), :], buf.at[0], sem_in)
    for k in range(NK):
        slot = k % 2
        r0 = base + k * C
        pltpu.make_async_copy(
            table_hbm.at[pl.ds(r0, C), :], buf.at[slot], sem_in
        ).wait()
        if k + 1 < NK:
            pltpu.async_copy(
                table_hbm.at[pl.ds(r0 + C, C), :], buf.at[1 - slot], sem_in
            )
        for b in range(B):
            pltpu.async_copy(
                buf.at[slot], out_hbm.at[b, pl.ds(r0, C), :], sem_out
            )
        for b in range(B):
            pltpu.make_async_copy(
                buf.at[slot], out_hbm.at[b, pl.ds(r0, C), :], sem_out
            ).wait()


def kernel(batch_size, table):
    del batch_size
    return _sc_broadcast(table)
